# Initial kernel scaffold; baseline (speedup 1.0000x reference)
import jax
import jax.numpy as jnp
from jax.experimental import pallas as pl


def kernel(x, edge_index, W_self, W_neigh, b):
    raise NotImplementedError("write your pallas kernel here")


# speedup vs baseline: 7.6412x; 7.6412x over previous
import functools

import jax
import jax.numpy as jnp
from jax import lax
from jax.experimental import pallas as pl
from jax.experimental.pallas import tpu as pltpu
from jax.experimental.pallas import tpu_sc as plsc

_NUM_CORES = 2
_NUM_SUBCORES = 16
_CHUNK = 512
_SCAT = 128


def _round_up(v, m):
    return (v + m - 1) // m * m


@functools.lru_cache(maxsize=None)
def _sc_aggregate(n_pad, d, e_pad):
    dh = d // 2
    per_tile = e_pad // _NUM_SUBCORES
    chunks = per_tile // _CHUNK
    sub = _CHUNK // _SCAT
    rows_per_sub = n_pad // _NUM_SUBCORES
    idx_rows_per_tile = per_tile // _SCAT

    mesh = plsc.VectorSubcoreMesh(core_axis_name="c", subcore_axis_name="s",
                                  num_cores=_NUM_CORES,
                                  num_subcores=_NUM_SUBCORES)

    @functools.partial(
        pl.kernel,
        compiler_params=pltpu.CompilerParams(use_tc_tiling_on_sc=False),
        out_type=[
            jax.ShapeDtypeStruct((_NUM_CORES, n_pad, dh), jnp.float32),
            jax.ShapeDtypeStruct((n_pad, 16), jnp.float32),
        ],
        mesh=mesh,
        scratch_types=[
            pltpu.VMEM((_CHUNK,), jnp.int32),
            pltpu.VMEM((sub, _SCAT), jnp.int32),
            pltpu.VMEM((_CHUNK, dh), jnp.float32),
            pltpu.VMEM((_SCAT, 16), jnp.float32),
            pltpu.VMEM_SHARED((n_pad, dh), jnp.float32),
            pltpu.VMEM_SHARED((n_pad, 16), jnp.float32),
        ],
    )
    def agg(x2_hbm, src_hbm, dst2_hbm, zrow_hbm, zcnt_hbm, ones_hbm,
            part_hbm, cnt_hbm, src_v, dst_v, rows_v, ones_v, acc_sh, cnt_sh):
        i32 = lambda v: jnp.int32(v)
        c = lax.convert_element_type(lax.axis_index("c"), jnp.int32)
        s = lax.convert_element_type(lax.axis_index("s"), jnp.int32)
        row0 = s * i32(rows_per_sub)

        pltpu.sync_copy(zrow_hbm, acc_sh.at[pl.ds(row0, rows_per_sub)])
        pltpu.sync_copy(zcnt_hbm, cnt_sh.at[pl.ds(row0, rows_per_sub)])
        pltpu.sync_copy(ones_hbm, ones_v)
        plsc.subcore_barrier()

        ebase = s * i32(per_tile)
        rbase = s * i32(idx_rows_per_tile)

        @pl.loop(0, chunks)
        def _(i):
            i = lax.convert_element_type(i, jnp.int32)
            pltpu.sync_copy(src_hbm.at[pl.ds(ebase + i * i32(_CHUNK), _CHUNK)],
                            src_v)
            pltpu.sync_copy(dst2_hbm.at[pl.ds(rbase + i * i32(sub), sub)],
                            dst_v)
            pltpu.sync_copy(x2_hbm.at[c].at[src_v], rows_v)
            for j in range(sub):
                pltpu.sync_copy(rows_v.at[pl.ds(j * _SCAT, _SCAT)],
                                acc_sh.at[dst_v.at[j]], add=True)

            @pl.when(c == 0)
            def _():
                for j in range(sub):
                    pltpu.sync_copy(ones_v, cnt_sh.at[dst_v.at[j]], add=True)

        plsc.subcore_barrier()
        pltpu.sync_copy(acc_sh.at[pl.ds(row0, rows_per_sub)],
                        part_hbm.at[c, pl.ds(row0, rows_per_sub)])

        @pl.when(c == 0)
        def _():
            pltpu.sync_copy(cnt_sh.at[pl.ds(row0, rows_per_sub)],
                            cnt_hbm.at[pl.ds(row0, rows_per_sub)])

    return agg


def _tc_body(x_ref, p_ref, c_ref, ws_ref, wn_ref, b_ref, o_ref):
    deg = jnp.maximum(c_ref[:, 0:1], 1.0)
    dh = p_ref.shape[2]
    h0 = p_ref[0] / deg
    h1 = p_ref[1] / deg
    dn = (((1,), (0,)), ((), ()))
    hp = lax.Precision.HIGHEST
    acc = lax.dot_general(x_ref[...], ws_ref[...], dn, precision=hp,
                          preferred_element_type=jnp.float32)
    acc = acc + lax.dot_general(h0, wn_ref[0:dh], dn, precision=hp,
                                preferred_element_type=jnp.float32)
    acc = acc + lax.dot_general(h1, wn_ref[dh:2 * dh], dn, precision=hp,
                                preferred_element_type=jnp.float32)
    o_ref[...] = jnp.maximum(acc + b_ref[...], 0.0)


def kernel(x, edge_index, W_self, W_neigh, b):
    with jax.enable_x64(False):
        return _kernel_32(x, edge_index, W_self, W_neigh, b)


def _kernel_32(x, edge_index, W_self, W_neigh, b):
    n, d = x.shape
    e = edge_index.shape[1]
    dh = d // 2

    n_pad = _round_up(n + 1, _NUM_SUBCORES * 64)
    e_pad = _round_up(e, _NUM_SUBCORES * _CHUNK)
    n_dummy = n_pad - n
    pad = e_pad - e

    src = edge_index[0].astype(jnp.int32)
    dst = edge_index[1].astype(jnp.int32)
    pad_idx = (n + jnp.arange(pad, dtype=jnp.int32) % n_dummy)
    src_p = jnp.concatenate([src, pad_idx])
    dst2 = jnp.concatenate([dst, pad_idx]).reshape(e_pad // _SCAT, _SCAT)

    x_pad = jnp.concatenate([x, jnp.zeros((n_dummy, d), jnp.float32)])
    x2 = jnp.stack([x_pad[:, :dh], x_pad[:, dh:]])
    rows_per_sub = n_pad // _NUM_SUBCORES
    zrow = jnp.zeros((rows_per_sub, dh), jnp.float32)
    zcnt = jnp.zeros((rows_per_sub, 16), jnp.float32)
    ones = jnp.ones((_SCAT, 16), jnp.float32)

    part, cnt = _sc_aggregate(n_pad, d, e_pad)(
        x2, src_p, dst2, zrow, zcnt, ones)

    bl = 1000
    grid = (n // bl,)
    out = pl.pallas_call(
        _tc_body,
        grid=grid,
        in_specs=[
            pl.BlockSpec((bl, d), lambda i: (i, 0)),
            pl.BlockSpec((_NUM_CORES, bl, dh), lambda i: (0, i, 0)),
            pl.BlockSpec((bl, 16), lambda i: (i, 0)),
            pl.BlockSpec((d, d), lambda i: (0, 0)),
            pl.BlockSpec((d, d), lambda i: (0, 0)),
            pl.BlockSpec((1, d), lambda i: (0, 0)),
        ],
        out_specs=pl.BlockSpec((bl, d), lambda i: (i, 0)),
        out_shape=jax.ShapeDtypeStruct((n, d), jnp.float32),
    )(x, part, cnt, W_self, W_neigh, b.reshape(1, d).astype(jnp.float32))
    return out


# speedup vs baseline: 9.7044x; 1.2700x over previous
import functools

import jax
import jax.numpy as jnp
from jax import lax
from jax.experimental import pallas as pl
from jax.experimental.pallas import tpu as pltpu
from jax.experimental.pallas import tpu_sc as plsc

_NUM_CORES = 2
_NUM_SUBCORES = 16
_CHUNK = 512
_SCAT = 128


def _round_up(v, m):
    return (v + m - 1) // m * m


@functools.lru_cache(maxsize=None)
def _sc_aggregate(n_pad, d, e_pad):
    dh = d // 2
    per_tile = e_pad // _NUM_SUBCORES
    chunks = per_tile // _CHUNK
    half_chunks = chunks // 2
    sub = _CHUNK // _SCAT
    rows_per_sub = n_pad // _NUM_SUBCORES
    idx_rows_per_tile = per_tile // _SCAT

    mesh = plsc.VectorSubcoreMesh(core_axis_name="c", subcore_axis_name="s",
                                  num_cores=_NUM_CORES,
                                  num_subcores=_NUM_SUBCORES)

    @functools.partial(
        pl.kernel,
        compiler_params=pltpu.CompilerParams(use_tc_tiling_on_sc=False),
        out_type=[
            jax.ShapeDtypeStruct((_NUM_CORES, n_pad, dh), jnp.float32),
            jax.ShapeDtypeStruct((_NUM_CORES, n_pad, 16), jnp.float32),
        ],
        mesh=mesh,
        scratch_types=[
            pltpu.VMEM((2, _CHUNK), jnp.int32),
            pltpu.VMEM((2 * sub, _SCAT), jnp.int32),
            pltpu.VMEM((2, _CHUNK, dh), jnp.float32),
            pltpu.VMEM((_SCAT, 16), jnp.float32),
            pltpu.VMEM_SHARED((n_pad, dh), jnp.float32),
            pltpu.VMEM_SHARED((n_pad, 16), jnp.float32),
            pltpu.SemaphoreType.DMA,
            pltpu.SemaphoreType.DMA,
        ],
    )
    def agg(x2_hbm, src_hbm, dst2_hbm, zrow_hbm, zcnt_hbm, ones_hbm,
            part_hbm, cnt_hbm, src_v, dst_v, rows_v, ones_v, acc_sh, cnt_sh,
            gsem0, gsem1):
        i32 = lambda v: jnp.int32(v)
        c = lax.convert_element_type(lax.axis_index("c"), jnp.int32)
        s = lax.convert_element_type(lax.axis_index("s"), jnp.int32)
        row0 = s * i32(rows_per_sub)

        pltpu.sync_copy(zrow_hbm, acc_sh.at[pl.ds(row0, rows_per_sub)])
        pltpu.sync_copy(zcnt_hbm, cnt_sh.at[pl.ds(row0, rows_per_sub)])
        pltpu.sync_copy(ones_hbm, ones_v)
        plsc.subcore_barrier()

        ebase = s * i32(per_tile)
        rbase = s * i32(idx_rows_per_tile)
        bufs = (
            (src_v.at[0], dst_v.at[pl.ds(0, sub)], rows_v.at[0], gsem0),
            (src_v.at[1], dst_v.at[pl.ds(sub, sub)], rows_v.at[1], gsem1),
        )

        def load_and_gather(chunk_i, k):
            sv, dv, rv, sem = bufs[k]
            pltpu.sync_copy(
                src_hbm.at[pl.ds(ebase + chunk_i * i32(_CHUNK), _CHUNK)], sv)
            pltpu.sync_copy(
                dst2_hbm.at[pl.ds(rbase + chunk_i * i32(sub), sub)], dv)
            pltpu.async_copy(x2_hbm.at[c].at[sv], rv, sem)

        def drain_and_scatter(chunk_i, k):
            sv, dv, rv, sem = bufs[k]
            pltpu.make_async_copy(x2_hbm.at[c].at[sv], rv, sem).wait()
            for j in range(sub):
                pltpu.sync_copy(rv.at[pl.ds(j * _SCAT, _SCAT)],
                                acc_sh.at[dv.at[j]], add=True)
            count_here = (chunk_i < i32(half_chunks)) == (c == 0)

            @pl.when(count_here)
            def _():
                for j in range(sub):
                    pltpu.sync_copy(ones_v, cnt_sh.at[dv.at[j]], add=True)

        load_and_gather(i32(0), 0)
        load_and_gather(i32(1), 1)

        @pl.loop(0, half_chunks)
        def _(t):
            t = lax.convert_element_type(t, jnp.int32)
            a = t * i32(2)
            drain_and_scatter(a, 0)

            @pl.when(a + i32(2) < i32(chunks))
            def _():
                load_and_gather(a + i32(2), 0)

            drain_and_scatter(a + i32(1), 1)

            @pl.when(a + i32(3) < i32(chunks))
            def _():
                load_and_gather(a + i32(3), 1)

        plsc.subcore_barrier()
        pltpu.sync_copy(acc_sh.at[pl.ds(row0, rows_per_sub)],
                        part_hbm.at[c, pl.ds(row0, rows_per_sub)])
        pltpu.sync_copy(cnt_sh.at[pl.ds(row0, rows_per_sub)],
                        cnt_hbm.at[c, pl.ds(row0, rows_per_sub)])

    return agg


def _tc_self_body(x_ref, ws_ref, b_ref, o_ref):
    hp = lax.Precision.HIGHEST
    dn = (((1,), (0,)), ((), ()))
    o_ref[...] = lax.dot_general(x_ref[...], ws_ref[...], dn, precision=hp,
                                 preferred_element_type=jnp.float32) + b_ref[...]


def _tc_finish_body(xw_ref, p_ref, c_ref, wn_ref, o_ref):
    deg = jnp.maximum(c_ref[0, :, 0:1] + c_ref[1, :, 0:1], 1.0)
    dh = p_ref.shape[2]
    h0 = p_ref[0] / deg
    h1 = p_ref[1] / deg
    dn = (((1,), (0,)), ((), ()))
    hp = lax.Precision.HIGHEST
    acc = xw_ref[...]
    acc = acc + lax.dot_general(h0, wn_ref[0:dh], dn, precision=hp,
                                preferred_element_type=jnp.float32)
    acc = acc + lax.dot_general(h1, wn_ref[dh:2 * dh], dn, precision=hp,
                                preferred_element_type=jnp.float32)
    o_ref[...] = jnp.maximum(acc, 0.0)


def kernel(x, edge_index, W_self, W_neigh, b):
    with jax.enable_x64(False):
        return _kernel_32(x, edge_index, W_self, W_neigh, b)


def _kernel_32(x, edge_index, W_self, W_neigh, b):
    n, d = x.shape
    e = edge_index.shape[1]
    dh = d // 2

    n_pad = _round_up(n + 1, _NUM_SUBCORES * 64)
    e_pad = _round_up(e, _NUM_SUBCORES * 2 * _CHUNK)
    n_dummy = n_pad - n
    pad = e_pad - e

    src = edge_index[0].astype(jnp.int32)
    dst = edge_index[1].astype(jnp.int32)
    pad_idx = (n + jnp.arange(pad, dtype=jnp.int32) % n_dummy)
    src_p = jnp.concatenate([src, pad_idx])
    dst2 = jnp.concatenate([dst, pad_idx]).reshape(e_pad // _SCAT, _SCAT)

    x_pad = jnp.concatenate([x, jnp.zeros((n_dummy, d), jnp.float32)])
    x2 = jnp.stack([x_pad[:, :dh], x_pad[:, dh:]])
    rows_per_sub = n_pad // _NUM_SUBCORES
    zrow = jnp.zeros((rows_per_sub, dh), jnp.float32)
    zcnt = jnp.zeros((rows_per_sub, 16), jnp.float32)
    ones = jnp.ones((_SCAT, 16), jnp.float32)

    part, cnt = _sc_aggregate(n_pad, d, e_pad)(
        x2, src_p, dst2, zrow, zcnt, ones)

    bl = 1000
    grid = (n // bl,)
    xw = pl.pallas_call(
        _tc_self_body,
        grid=grid,
        in_specs=[
            pl.BlockSpec((bl, d), lambda i: (i, 0)),
            pl.BlockSpec((d, d), lambda i: (0, 0)),
            pl.BlockSpec((1, d), lambda i: (0, 0)),
        ],
        out_specs=pl.BlockSpec((bl, d), lambda i: (i, 0)),
        out_shape=jax.ShapeDtypeStruct((n, d), jnp.float32),
    )(x, W_self, b.reshape(1, d).astype(jnp.float32))

    out = pl.pallas_call(
        _tc_finish_body,
        grid=grid,
        in_specs=[
            pl.BlockSpec((bl, d), lambda i: (i, 0)),
            pl.BlockSpec((_NUM_CORES, bl, dh), lambda i: (0, i, 0)),
            pl.BlockSpec((_NUM_CORES, bl, 16), lambda i: (0, i, 0)),
            pl.BlockSpec((d, d), lambda i: (0, 0)),
        ],
        out_specs=pl.BlockSpec((bl, d), lambda i: (i, 0)),
        out_shape=jax.ShapeDtypeStruct((n, d), jnp.float32),
    )(xw, part, cnt, W_neigh)
    return out


# speedup vs baseline: 11.4038x; 1.1751x over previous
import functools

import jax
import jax.numpy as jnp
from jax import lax
from jax.experimental import pallas as pl
from jax.experimental.pallas import tpu as pltpu
from jax.experimental.pallas import tpu_sc as plsc

_NUM_CORES = 2
_NUM_SUBCORES = 16
_CHUNK = 400


@functools.lru_cache(maxsize=None)
def _sc_aggregate(n, d, e):
    dh = d // 2
    per_tile = e // _NUM_SUBCORES
    chunks = per_tile // _CHUNK
    half_chunks = chunks // 2
    rows_per_sub = n // _NUM_SUBCORES

    mesh = plsc.VectorSubcoreMesh(core_axis_name="c", subcore_axis_name="s",
                                  num_cores=_NUM_CORES,
                                  num_subcores=_NUM_SUBCORES)

    @functools.partial(
        pl.kernel,
        compiler_params=pltpu.CompilerParams(use_tc_tiling_on_sc=False),
        out_type=[
            jax.ShapeDtypeStruct((n, d), jnp.float32),
            jax.ShapeDtypeStruct((_NUM_CORES, n, 16), jnp.float32),
        ],
        mesh=mesh,
        scratch_types=[
            pltpu.VMEM((2, _CHUNK), jnp.int32),
            pltpu.VMEM((2, _CHUNK), jnp.int32),
            pltpu.VMEM((2, _CHUNK, dh), jnp.float32),
            pltpu.VMEM((_CHUNK, 16), jnp.float32),
            pltpu.VMEM_SHARED((n, dh), jnp.float32),
            pltpu.VMEM_SHARED((n, 16), jnp.float32),
            pltpu.SemaphoreType.DMA,
            pltpu.SemaphoreType.DMA,
        ],
    )
    def agg(xr_hbm, src2_hbm, dst_hbm, zrow_hbm, zcnt_hbm, ones_hbm,
            part_hbm, cnt_hbm, src_v, dst_v, rows_v, ones_v, acc_sh, cnt_sh,
            gsem0, gsem1):
        i32 = lambda v: jnp.int32(v)
        c = lax.convert_element_type(lax.axis_index("c"), jnp.int32)
        s = lax.convert_element_type(lax.axis_index("s"), jnp.int32)
        row0 = s * i32(rows_per_sub)

        pltpu.sync_copy(zrow_hbm, acc_sh.at[pl.ds(row0, rows_per_sub)])
        pltpu.sync_copy(zcnt_hbm, cnt_sh.at[pl.ds(row0, rows_per_sub)])
        pltpu.sync_copy(ones_hbm, ones_v)
        plsc.subcore_barrier()

        ebase = s * i32(per_tile)
        bufs = (
            (src_v.at[0], dst_v.at[0], rows_v.at[0], gsem0),
            (src_v.at[1], dst_v.at[1], rows_v.at[1], gsem1),
        )

        def load_and_gather(chunk_i, k):
            sv, dv, rv, sem = bufs[k]
            off = ebase + chunk_i * i32(_CHUNK)
            pltpu.sync_copy(src2_hbm.at[c, pl.ds(off, _CHUNK)], sv)
            pltpu.sync_copy(dst_hbm.at[pl.ds(off, _CHUNK)], dv)
            pltpu.async_copy(xr_hbm.at[sv], rv, sem)

        def drain_and_scatter(chunk_i, k):
            sv, dv, rv, sem = bufs[k]
            pltpu.make_async_copy(xr_hbm.at[sv], rv, sem).wait()
            pltpu.sync_copy(rv, acc_sh.at[dv], add=True)
            count_here = (chunk_i < i32(half_chunks)) == (c == 0)

            @pl.when(count_here)
            def _():
                pltpu.sync_copy(ones_v, cnt_sh.at[dv], add=True)

        load_and_gather(i32(0), 0)
        load_and_gather(i32(1), 1)

        @pl.loop(0, half_chunks)
        def _(t):
            t = lax.convert_element_type(t, jnp.int32)
            a = t * i32(2)
            drain_and_scatter(a, 0)

            @pl.when(a + i32(2) < i32(chunks))
            def _():
                load_and_gather(a + i32(2), 0)

            drain_and_scatter(a + i32(1), 1)

            @pl.when(a + i32(3) < i32(chunks))
            def _():
                load_and_gather(a + i32(3), 1)

        plsc.subcore_barrier()
        pltpu.sync_copy(acc_sh.at[pl.ds(row0, rows_per_sub)],
                        part_hbm.at[pl.ds(row0, rows_per_sub),
                                    pl.ds(c * i32(dh), dh)])
        pltpu.sync_copy(cnt_sh.at[pl.ds(row0, rows_per_sub)],
                        cnt_hbm.at[c, pl.ds(row0, rows_per_sub)])

    return agg


def _tc_self_body(x_ref, ws_ref, b_ref, o_ref):
    hp = lax.Precision.HIGHEST
    dn = (((1,), (0,)), ((), ()))
    o_ref[...] = lax.dot_general(x_ref[...], ws_ref[...], dn, precision=hp,
                                 preferred_element_type=jnp.float32) + b_ref[...]


def _tc_finish_body(xw_ref, p_ref, c_ref, wn_ref, o_ref):
    deg = jnp.maximum(c_ref[0, :, 0:1] + c_ref[1, :, 0:1], 1.0)
    h = p_ref[...] / deg
    dn = (((1,), (0,)), ((), ()))
    acc = lax.dot_general(h, wn_ref[...], dn, precision=lax.Precision.HIGHEST,
                          preferred_element_type=jnp.float32)
    o_ref[...] = jnp.maximum(acc + xw_ref[...], 0.0)


def kernel(x, edge_index, W_self, W_neigh, b):
    with jax.enable_x64(False):
        return _kernel_32(x, edge_index, W_self, W_neigh, b)


def _kernel_32(x, edge_index, W_self, W_neigh, b):
    n, d = x.shape
    e = edge_index.shape[1]
    dh = d // 2

    ei32 = lax.bitcast_convert_type(edge_index, jnp.int32)
    src = ei32[0, :, 0]
    dst = ei32[1, :, 0]
    src2 = jnp.stack([2 * src, 2 * src + 1])

    xr = jnp.reshape(x, (2 * n, dh))
    rows_per_sub = n // _NUM_SUBCORES
    zrow = jnp.zeros((rows_per_sub, dh), jnp.float32)
    zcnt = jnp.zeros((rows_per_sub, 16), jnp.float32)
    ones = jnp.ones((_CHUNK, 16), jnp.float32)

    part, cnt = _sc_aggregate(n, d, e)(xr, src2, dst, zrow, zcnt, ones)

    bl = 1000
    grid = (n // bl,)
    xw = pl.pallas_call(
        _tc_self_body,
        grid=grid,
        in_specs=[
            pl.BlockSpec((bl, d), lambda i: (i, 0)),
            pl.BlockSpec((d, d), lambda i: (0, 0)),
            pl.BlockSpec((1, d), lambda i: (0, 0)),
        ],
        out_specs=pl.BlockSpec((bl, d), lambda i: (i, 0)),
        out_shape=jax.ShapeDtypeStruct((n, d), jnp.float32),
    )(x, W_self, b.reshape(1, d).astype(jnp.float32))

    out = pl.pallas_call(
        _tc_finish_body,
        grid=grid,
        in_specs=[
            pl.BlockSpec((bl, d), lambda i: (i, 0)),
            pl.BlockSpec((bl, d), lambda i: (i, 0)),
            pl.BlockSpec((_NUM_CORES, bl, 16), lambda i: (0, i, 0)),
            pl.BlockSpec((d, d), lambda i: (0, 0)),
        ],
        out_specs=pl.BlockSpec((bl, d), lambda i: (i, 0)),
        out_shape=jax.ShapeDtypeStruct((n, d), jnp.float32),
    )(xw, part, cnt, W_neigh)
    return out
